# Initial kernel scaffold; baseline (speedup 1.0000x reference)
import jax
import jax.numpy as jnp
from jax.experimental import pallas as pl


def kernel(features, edge_index, W1, b1, W2, b2):
    raise NotImplementedError("write your pallas kernel here")


# speedup vs baseline: 4.7913x; 4.7913x over previous
import functools

import jax
import jax.numpy as jnp
from jax import lax
from jax.experimental import pallas as pl
from jax.experimental.pallas import tpu as pltpu
from jax.experimental.pallas import tpu_sc as plsc

NC = 2
NS = 16
NW = NC * NS
CH = 80
SR = CH
DW = 16


def _npad(n):
    return -(-n // (NS * SR)) * NS * SR


def _fill_loop(ref, nrows, d, value):
    vv = jnp.full((16,), value, jnp.float32)

    def frow(i, carry):
        for j in range(d // 16):
            ref[i, pl.ds(16 * j, 16)] = vv
        return carry

    lax.fori_loop(0, nrows, frow, 0)


@functools.lru_cache(maxsize=None)
def _sc_aggregate(n, e, d, with_gather):
    epw = e // NW
    nchunk = epw // CH
    assert e % NW == 0 and epw % CH == 0 and d % 16 == 0
    npad = _npad(n)
    rpt = npad // NS
    nstage = rpt // SR

    mesh = plsc.VectorSubcoreMesh(core_axis_name="c", subcore_axis_name="s")

    @functools.partial(
        pl.kernel, mesh=mesh,
        out_type=jax.ShapeDtypeStruct((NC * npad, d), jnp.float32),
        scratch_types=[
            pltpu.VMEM_SHARED((npad, d), jnp.float32),
            pltpu.VMEM((CH,), jnp.int32),
            pltpu.VMEM((CH,), jnp.int32),
            pltpu.VMEM((CH, d), jnp.float32),
            pltpu.SemaphoreType.DMA,
        ],
    )
    def agg(h_hbm, src_hbm, dst_hbm, p_out, acc, src_v, dst_v, rows_v, sem):
        cid = lax.axis_index("c")
        sid = lax.axis_index("s")
        wid = sid * NC + cid
        row0 = sid * rpt

        _fill_loop(rows_v, SR, d, 0.0)
        for k in range(nstage):
            pltpu.sync_copy(rows_v, acc.at[pl.ds(row0 + SR * k, SR)])
        if not with_gather:
            _fill_loop(rows_v, CH, d, 1.0)
        plsc.subcore_barrier()

        base0 = wid * epw

        def body(c, carry):
            base = base0 + c * CH
            pltpu.sync_copy(dst_hbm.at[pl.ds(base, CH)], dst_v)
            if with_gather:
                pltpu.sync_copy(src_hbm.at[pl.ds(base, CH)], src_v)
                pltpu.async_copy(h_hbm.at[src_v], rows_v, sem).wait()
            pltpu.sync_copy(rows_v, acc.at[dst_v], add=True)
            return carry

        lax.fori_loop(0, nchunk, body, 0)
        plsc.subcore_barrier()

        out0 = cid * npad + row0
        for k in range(nstage):
            pltpu.sync_copy(acc.at[pl.ds(row0 + SR * k, SR)], rows_v)
            pltpu.sync_copy(rows_v, p_out.at[pl.ds(out0 + SR * k, SR)])

    return agg


def _dense_body(relu, pp, dp, h, w, b, o):
    s = pp[0] + pp[1] + h[...]
    deg = dp[0][:, 0:1] + dp[1][:, 0:1]
    hn = s / (deg + 1.0)
    y = jnp.dot(hn, w[...], preferred_element_type=jnp.float32) + b[...]
    if relu:
        y = jnp.maximum(y, 0.0)
    o[...] = y


@functools.lru_cache(maxsize=None)
def _dense(n, npad, d_in, d_out, relu, bn=1000):
    assert n % bn == 0
    grid = (n // bn,)
    return pl.pallas_call(
        functools.partial(_dense_body, relu),
        grid=grid,
        in_specs=[
            pl.BlockSpec((NC, bn, d_in), lambda i: (0, i, 0)),
            pl.BlockSpec((NC, bn, DW), lambda i: (0, i, 0)),
            pl.BlockSpec((bn, d_in), lambda i: (i, 0)),
            pl.BlockSpec((d_in, d_out), lambda i: (0, 0)),
            pl.BlockSpec((d_out,), lambda i: (0,)),
        ],
        out_specs=pl.BlockSpec((bn, d_out), lambda i: (i, 0)),
        out_shape=jax.ShapeDtypeStruct((n, d_out), jnp.float32),
    )


def _unwrap(x):
    return x[0] if isinstance(x, (list, tuple)) else x


def kernel(features, edge_index, W1, b1, W2, b2):
    n, d = features.shape
    e = edge_index.shape[1]
    src = edge_index[0]
    dst = edge_index[1]
    npad = _npad(n)

    p1 = _unwrap(_sc_aggregate(n, e, d, True)(features, src, dst))
    p1 = p1.reshape(NC, npad, d)
    pd = _unwrap(_sc_aggregate(n, e, d, False)(features, src, dst))
    degp = pd.reshape(NC, npad, d)[:, :, :DW]
    h1 = _dense(n, npad, d, W1.shape[1], True)(p1, degp, features, W1, b1)
    p2 = _unwrap(_sc_aggregate(n, e, d, True)(h1, src, dst))
    p2 = p2.reshape(NC, npad, d)
    h2 = _dense(n, npad, W1.shape[1], W2.shape[1], False)(p2, degp, h1, W2, b2)
    return (features, h1, h2)


# speedup vs baseline: 5.9458x; 1.2410x over previous
import functools

import jax
import jax.numpy as jnp
from jax import lax
from jax.experimental import pallas as pl
from jax.experimental.pallas import tpu as pltpu
from jax.experimental.pallas import tpu_sc as plsc

NC = 2
NS = 16
NW = NC * NS
CH = 80
SR = CH
DW = 16


def _npad(n):
    return -(-n // (NS * SR)) * NS * SR


def _fill_loop(ref, nrows, d, value):
    vv = jnp.full((16,), value, jnp.float32)

    def frow(i, carry):
        for j in range(d // 16):
            ref[i, pl.ds(16 * j, 16)] = vv
        return carry

    lax.fori_loop(0, nrows, frow, 0)


@functools.lru_cache(maxsize=None)
def _sc_aggregate(n, e, d, with_gather):
    epw = e // NW
    nchunk = epw // CH
    assert e % NW == 0 and epw % CH == 0 and d % 16 == 0
    npad = _npad(n)
    rpt = npad // NS
    nstage = rpt // SR

    mesh = plsc.VectorSubcoreMesh(core_axis_name="c", subcore_axis_name="s")

    scratch = [
        pltpu.VMEM_SHARED((npad, d), jnp.float32),
        pltpu.VMEM((CH,), jnp.int32),
        pltpu.VMEM((CH,), jnp.int32),
        pltpu.VMEM((CH, d), jnp.float32),
        pltpu.SemaphoreType.DMA,
    ]
    if with_gather:
        scratch += [
            pltpu.VMEM((CH,), jnp.int32),
            pltpu.VMEM((CH,), jnp.int32),
            pltpu.VMEM((CH, d), jnp.float32),
        ]

    @functools.partial(pl.kernel, mesh=mesh,
                       out_type=jax.ShapeDtypeStruct((NC * npad, d),
                                                     jnp.float32),
                       scratch_types=scratch)
    def agg(h_hbm, src_hbm, dst_hbm, p_out, acc, src_v, dst_v, rows_v,
            sem, *rest):
        cid = lax.axis_index("c")
        sid = lax.axis_index("s")
        wid = sid * NC + cid
        row0 = sid * rpt

        _fill_loop(rows_v, SR, d, 0.0)
        for k in range(nstage):
            pltpu.sync_copy(rows_v, acc.at[pl.ds(row0 + SR * k, SR)])
        if not with_gather:
            _fill_loop(rows_v, CH, d, 1.0)
        plsc.subcore_barrier()

        base0 = wid * epw

        if with_gather:
            src_w, dst_w, rows_w = rest

            def body(i, carry):
                b0 = base0 + (2 * i) * CH
                b1 = b0 + CH
                pltpu.sync_copy(src_hbm.at[pl.ds(b0, CH)], src_v)
                pltpu.sync_copy(dst_hbm.at[pl.ds(b0, CH)], dst_v)
                cp0 = pltpu.async_copy(h_hbm.at[src_v], rows_v, sem)
                pltpu.sync_copy(src_hbm.at[pl.ds(b1, CH)], src_w)
                pltpu.sync_copy(dst_hbm.at[pl.ds(b1, CH)], dst_w)
                cp0.wait()
                cp1 = pltpu.async_copy(h_hbm.at[src_w], rows_w, sem)
                pltpu.sync_copy(rows_v, acc.at[dst_v], add=True)
                cp1.wait()
                pltpu.sync_copy(rows_w, acc.at[dst_w], add=True)
                return carry

            lax.fori_loop(0, nchunk // 2, body, 0)
            for c in range(2 * (nchunk // 2), nchunk):
                base = base0 + c * CH
                pltpu.sync_copy(src_hbm.at[pl.ds(base, CH)], src_v)
                pltpu.sync_copy(dst_hbm.at[pl.ds(base, CH)], dst_v)
                pltpu.async_copy(h_hbm.at[src_v], rows_v, sem).wait()
                pltpu.sync_copy(rows_v, acc.at[dst_v], add=True)
        else:
            def body(c, carry):
                base = base0 + c * CH
                pltpu.sync_copy(dst_hbm.at[pl.ds(base, CH)], dst_v)
                pltpu.sync_copy(rows_v, acc.at[dst_v], add=True)
                return carry

            lax.fori_loop(0, nchunk, body, 0)
        plsc.subcore_barrier()

        out0 = cid * npad + row0
        for k in range(nstage):
            pltpu.sync_copy(acc.at[pl.ds(row0 + SR * k, SR)], rows_v)
            pltpu.sync_copy(rows_v, p_out.at[pl.ds(out0 + SR * k, SR)])

    return agg


def _dense_body(relu, pp, dp, h, w, b, o):
    s = pp[0] + pp[1] + h[...]
    deg = dp[0][:, 0:1] + dp[1][:, 0:1]
    hn = s / (deg + 1.0)
    y = jnp.dot(hn, w[...], preferred_element_type=jnp.float32) + b[...]
    if relu:
        y = jnp.maximum(y, 0.0)
    o[...] = y


@functools.lru_cache(maxsize=None)
def _dense(n, npad, d_in, d_out, relu, bn=1000):
    assert n % bn == 0
    grid = (n // bn,)
    return pl.pallas_call(
        functools.partial(_dense_body, relu),
        grid=grid,
        in_specs=[
            pl.BlockSpec((NC, bn, d_in), lambda i: (0, i, 0)),
            pl.BlockSpec((NC, bn, DW), lambda i: (0, i, 0)),
            pl.BlockSpec((bn, d_in), lambda i: (i, 0)),
            pl.BlockSpec((d_in, d_out), lambda i: (0, 0)),
            pl.BlockSpec((d_out,), lambda i: (0,)),
        ],
        out_specs=pl.BlockSpec((bn, d_out), lambda i: (i, 0)),
        out_shape=jax.ShapeDtypeStruct((n, d_out), jnp.float32),
    )


def _unwrap(x):
    return x[0] if isinstance(x, (list, tuple)) else x


def kernel(features, edge_index, W1, b1, W2, b2):
    n, d = features.shape
    e = edge_index.shape[1]
    src = edge_index[0]
    dst = edge_index[1]
    npad = _npad(n)

    p1 = _unwrap(_sc_aggregate(n, e, d, True)(features, src, dst))
    p1 = p1.reshape(NC, npad, d)
    pd = _unwrap(_sc_aggregate(n, e, d, False)(features, src, dst))
    degp = pd.reshape(NC, npad, d)[:, :, :DW]
    h1 = _dense(n, npad, d, W1.shape[1], True)(p1, degp, features, W1, b1)
    p2 = _unwrap(_sc_aggregate(n, e, d, True)(h1, src, dst))
    p2 = p2.reshape(NC, npad, d)
    h2 = _dense(n, npad, W1.shape[1], W2.shape[1], False)(p2, degp, h1, W2, b2)
    return (features, h1, h2)
